# baseline (speedup 1.0000x reference)
import functools

import jax
import jax.numpy as jnp
from jax import lax
from jax.experimental import pallas as pl
from jax.experimental.pallas import tpu as pltpu
from jax.experimental.pallas import tpu_sc as plsc

T = 2048
H = 768
FF = 3072
E = 8
K = 2

BLK = 256
NB = (T * K + E * BLK) // BLK
N_PAD = NB * BLK
FF_BLK = 768
NF = FF // FF_BLK

NW = 32
CT = T // NW


def _router_body(x_ref, gw_ref, slots_ref, wts_ref, be_ref):
    x = x_ref[...]
    gw = gw_ref[...]
    logits = lax.dot_general(x, gw, (((1,), (1,)), ((), ())),
                             preferred_element_type=jnp.float32)
    e_iota = lax.broadcasted_iota(jnp.int32, (T, E), 1)
    m1 = jnp.max(logits, axis=1, keepdims=True)
    idx1 = jnp.min(jnp.where(logits == m1, e_iota, E), axis=1, keepdims=True)
    masked = jnp.where(e_iota == idx1, -1e30, logits)
    m2 = jnp.max(masked, axis=1, keepdims=True)
    idx2 = jnp.min(jnp.where(masked == m2, e_iota, E), axis=1, keepdims=True)
    d = jnp.exp(m2 - m1)
    w0 = 1.0 / (1.0 + d)
    w1 = d / (1.0 + d)

    oh0 = (e_iota == idx1).astype(jnp.float32)
    oh1 = (e_iota == idx2).astype(jnp.float32)
    ohs = oh0 + oh1
    r_i = lax.broadcasted_iota(jnp.int32, (T, T), 0)
    c_i = lax.broadcasted_iota(jnp.int32, (T, T), 1)
    strict_lt = (c_i < r_i).astype(jnp.float32)
    rank = lax.dot_general(strict_lt, ohs, (((1,), (0,)), ((), ())),
                           preferred_element_type=jnp.float32)
    counts = rank[T - 1:T, :] + ohs[T - 1:T, :]
    padded = jnp.ceil(counts * (1.0 / BLK)) * BLK
    lt8_r = lax.broadcasted_iota(jnp.int32, (E, E), 0)
    lt8_c = lax.broadcasted_iota(jnp.int32, (E, E), 1)
    lt8 = (lt8_r < lt8_c).astype(jnp.float32)
    off = lax.dot_general(padded, lt8, (((1,), (0,)), ((), ())),
                          preferred_element_type=jnp.float32)
    off_end = off + padded

    slot0 = jnp.sum((off + rank) * oh0, axis=1)[None, :]
    slot1 = jnp.sum((off + rank) * oh1, axis=1)[None, :]
    slots_ref[...] = jnp.concatenate([slot0, slot1], axis=0).astype(jnp.int32)
    wts_ref[...] = jnp.concatenate([jnp.transpose(w0), jnp.transpose(w1)], axis=0)

    nb_i = (lax.broadcasted_iota(jnp.int32, (NB, E), 0) * BLK).astype(jnp.float32)
    be = jnp.sum((nb_i >= off_end).astype(jnp.int32), axis=1, keepdims=True)
    be = jnp.minimum(be, E - 1)

    r24 = lax.broadcasted_iota(jnp.int32, (NB, NB), 0)
    c24 = lax.broadcasted_iota(jnp.int32, (NB, NB), 1)
    be_r = jnp.broadcast_to(be, (NB, NB))
    be_c = jnp.broadcast_to(jnp.transpose(be), (NB, NB))
    tr_mat = jnp.where((c24 == r24 - 1) & (be_r != be_c), 1, 0)
    tr0 = jnp.sum(tr_mat, axis=1, keepdims=True)
    trc = jnp.broadcast_to(jnp.transpose(tr0), (NB, NB))
    tc = jnp.sum(jnp.where(c24 <= r24, trc, 0), axis=1, keepdims=True)
    parity = tc % 2
    ntm = jnp.where((c24 > r24) & (trc == 1), c24, NB)
    nt = jnp.min(ntm, axis=1, keepdims=True)
    nb_col = lax.broadcasted_iota(jnp.int32, (NB, 1), 0)
    isf = jnp.where((tr0 == 1) | (nb_col == 0), 1, 0)
    be_ref[...] = jnp.concatenate([be, parity, nt, isf], axis=1)


def _router(flat, gate_w):
    return pl.pallas_call(
        _router_body,
        out_shape=[
            jax.ShapeDtypeStruct((K, T), jnp.int32),
            jax.ShapeDtypeStruct((K, T), jnp.float32),
            jax.ShapeDtypeStruct((NB, 4), jnp.int32),
        ],
    )(flat, gate_w)


def _dispatch_body(flat_hbm, slots_hbm, wts_hbm,
                   xs_hbm, ws_hbm,
                   rows_v, idx0_v, idx1_v, slots_v, wts_v, wsort_v, sem):
    wid = lax.axis_index("s") * 2 + lax.axis_index("c")
    base = pl.multiple_of(wid * CT, CT)
    pltpu.sync_copy(flat_hbm.at[pl.ds(base, CT)], rows_v)
    pltpu.sync_copy(slots_hbm.at[0, pl.ds(base, CT)], idx0_v)
    pltpu.sync_copy(slots_hbm.at[1, pl.ds(base, CT)], idx1_v)
    cp0 = pltpu.async_copy(rows_v, xs_hbm.at[idx0_v], sem)
    cp1 = pltpu.async_copy(rows_v, xs_hbm.at[idx1_v], sem)

    @pl.when(wid == 0)
    def _():
        pltpu.sync_copy(slots_hbm, slots_v)
        pltpu.sync_copy(wts_hbm, wts_v)

        def zero(i, _):
            wsort_v[pl.ds(i * 16, 16)] = jnp.zeros((16,), jnp.float32)
            return 0
        lax.fori_loop(0, N_PAD // 16, zero, 0)

        def scat(i, _):
            k = i // (T // 16)
            j = i % (T // 16)
            sl = slots_v[k, pl.ds(j * 16, 16)]
            wv = wts_v[k, pl.ds(j * 16, 16)]
            plsc.store_scatter(wsort_v, [sl], wv)
            return 0
        lax.fori_loop(0, K * (T // 16), scat, 0)
        pltpu.sync_copy(wsort_v, ws_hbm)

    cp0.wait()
    cp1.wait()


def _dispatch(flat, slots, wts):
    mesh = plsc.VectorSubcoreMesh(core_axis_name="c", subcore_axis_name="s", num_cores=2, num_subcores=16)
    f = functools.partial(
        pl.kernel,
        out_type=[
            jax.ShapeDtypeStruct((N_PAD, H), jnp.float32),
            jax.ShapeDtypeStruct((N_PAD,), jnp.float32),
        ],
        mesh=mesh,
        scratch_types=[
            pltpu.VMEM((CT, H), jnp.float32),
            pltpu.VMEM((CT,), jnp.int32),
            pltpu.VMEM((CT,), jnp.int32),
            pltpu.VMEM((K, T), jnp.int32),
            pltpu.VMEM((K, T), jnp.float32),
            pltpu.VMEM((N_PAD,), jnp.float32),
            pltpu.SemaphoreType.DMA,
        ],
        compiler_params=pltpu.CompilerParams(needs_layout_passes=False),
    )(_dispatch_body)
    return f(flat, slots, wts)


def _ffn_body(be_ref, x_ref, demb_ref, w1_any, w3_any, w2_any, ws_ref, o_ref,
              w1b, w3b, w2b, sems):
    nb = pl.program_id(0)

    def copies(e, slot):
        hf = FF // 2
        return (
            pltpu.make_async_copy(w1_any.at[e, pl.ds(0, hf)],
                                  w1b.at[slot, pl.ds(0, hf)], sems.at[slot, 0]),
            pltpu.make_async_copy(w1_any.at[e, pl.ds(hf, hf)],
                                  w1b.at[slot, pl.ds(hf, hf)], sems.at[slot, 1]),
            pltpu.make_async_copy(w3_any.at[e, pl.ds(0, hf)],
                                  w3b.at[slot, pl.ds(0, hf)], sems.at[slot, 2]),
            pltpu.make_async_copy(w3_any.at[e, pl.ds(hf, hf)],
                                  w3b.at[slot, pl.ds(hf, hf)], sems.at[slot, 3]),
            pltpu.make_async_copy(w2_any.at[e, :, pl.ds(0, hf)],
                                  w2b.at[slot, :, pl.ds(0, hf)], sems.at[slot, 4]),
            pltpu.make_async_copy(w2_any.at[e, :, pl.ds(hf, hf)],
                                  w2b.at[slot, :, pl.ds(hf, hf)], sems.at[slot, 5]),
        )

    parity = be_ref[nb, 1]
    nt = be_ref[nb, 2]
    is_first = be_ref[nb, 3] == 1

    @pl.when(nb == 0)
    def _():
        for c in copies(be_ref[0, 0], 0):
            c.start()

    @pl.when(jnp.logical_and(is_first, nt < NB))
    def _():
        for c in copies(be_ref[jnp.minimum(nt, NB - 1), 0], (parity + 1) % 2):
            c.start()

    @pl.when(is_first)
    def _():
        for c in copies(be_ref[nb, 0], parity):
            c.wait()

    xb = x_ref[...] + demb_ref[0]
    h1 = lax.dot_general(xb, w1b[parity], (((1,), (1,)), ((), ())),
                         preferred_element_type=jnp.float32)
    h1 = h1 * jax.nn.sigmoid(h1)
    h2 = lax.dot_general(xb, w3b[parity], (((1,), (1,)), ((), ())),
                         preferred_element_type=jnp.float32)
    hh = h1 * h2
    part = lax.dot_general(hh, w2b[parity], (((1,), (1,)), ((), ())),
                           preferred_element_type=jnp.float32)
    w = ws_ref[0, 0, :][:, None]
    o_ref[...] = jnp.where(w == 0.0, 0.0, part * w)


def _ffn(block_expert, x_sorted, domain_emb, w1, w3, w2, w_sorted):
    ws3 = w_sorted.reshape(NB, 1, BLK)
    grid_spec = pltpu.PrefetchScalarGridSpec(
        num_scalar_prefetch=1,
        grid=(NB,),
        in_specs=[
            pl.BlockSpec((BLK, H), lambda nb, be: (nb, 0)),
            pl.BlockSpec((1, 1, H), lambda nb, be: (be[nb, 0], 0, 0)),
            pl.BlockSpec(memory_space=pl.ANY),
            pl.BlockSpec(memory_space=pl.ANY),
            pl.BlockSpec(memory_space=pl.ANY),
            pl.BlockSpec((1, 1, BLK), lambda nb, be: (nb, 0, 0)),
        ],
        out_specs=pl.BlockSpec((BLK, H), lambda nb, be: (nb, 0)),
        scratch_shapes=[
            pltpu.VMEM((2, FF, H), jnp.float32),
            pltpu.VMEM((2, FF, H), jnp.float32),
            pltpu.VMEM((2, H, FF), jnp.float32),
            pltpu.SemaphoreType.DMA((2, 6)),
        ],
    )
    return pl.pallas_call(
        _ffn_body,
        grid_spec=grid_spec,
        out_shape=jax.ShapeDtypeStruct((N_PAD, H), jnp.float32),
        compiler_params=pltpu.CompilerParams(vmem_limit_bytes=120 * 1024 * 1024),
    )(block_expert, x_sorted, domain_emb.reshape(E, 1, H), w1, w3, w2, ws3)


NCH = 4
CR = CT // NCH


def _combine_body(y_hbm, slots_hbm, out_hbm, y0_v, y1_v, i0_v, i1_v, sems):
    wid = lax.axis_index("s") * 2 + lax.axis_index("c")
    base = pl.multiple_of(wid * CT, CT)
    pltpu.sync_copy(slots_hbm.at[0, pl.ds(base, CT)], i0_v)
    pltpu.sync_copy(slots_hbm.at[1, pl.ds(base, CT)], i1_v)
    cps = []
    for c in range(NCH):
        sl = pl.ds(c * CR, CR)
        cps.append(pltpu.async_copy(
            y_hbm.at[i0_v.at[sl]], y0_v.at[sl], sems.at[c, 0]))
        cps.append(pltpu.async_copy(
            y_hbm.at[i1_v.at[sl]], y1_v.at[sl], sems.at[c, 1]))
    for c in range(NCH):
        cps[2 * c].wait()
        cps[2 * c + 1].wait()

        def add(i, _):
            r = c * CR + i // (H // 64)
            col = (i % (H // 64)) * 64
            for u in range(4):
                s = pl.ds(col + u * 16, 16)
                y0_v[r, s] = y0_v[r, s] + y1_v[r, s]
            return 0
        lax.fori_loop(0, CR * (H // 64), add, 0)
    pltpu.sync_copy(y0_v, out_hbm.at[pl.ds(base, CT)])


def _combine(y, slots):
    mesh = plsc.VectorSubcoreMesh(core_axis_name="c", subcore_axis_name="s", num_cores=2, num_subcores=16)
    f = functools.partial(
        pl.kernel,
        out_type=jax.ShapeDtypeStruct((T, H), jnp.float32),
        mesh=mesh,
        scratch_types=[
            pltpu.VMEM((CT, H), jnp.float32),
            pltpu.VMEM((CT, H), jnp.float32),
            pltpu.VMEM((CT,), jnp.int32),
            pltpu.VMEM((CT,), jnp.int32),
            pltpu.SemaphoreType.DMA((NCH, 2)),
        ],
        compiler_params=pltpu.CompilerParams(needs_layout_passes=False),
    )(_combine_body)
    return f(y, slots)


def kernel(hidden_states, gate_w, w1, w2, w3, domain_emb):
    b, s, h = hidden_states.shape
    flat = hidden_states.reshape(T, H)
    slots, wts, be = _router(flat, gate_w)
    x_sorted, w_sorted = _dispatch(flat, slots, wts)
    y = _ffn(be, x_sorted, domain_emb, w1, w3, w2, w_sorted)
    out = _combine(y, slots)
    return out.reshape(b, s, h)


# speedup vs baseline: 1.0215x; 1.0215x over previous
import functools

import jax
import jax.numpy as jnp
from jax import lax
from jax.experimental import pallas as pl
from jax.experimental.pallas import tpu as pltpu
from jax.experimental.pallas import tpu_sc as plsc

T = 2048
H = 768
FF = 3072
E = 8
K = 2

BLK = 256
NB = (T * K + E * BLK) // BLK
N_PAD = NB * BLK
FF_BLK = 768
NF = FF // FF_BLK

NW = 32
CT = T // NW


def _router_body(x_ref, gw_ref, slots_ref, wts_ref, be_ref):
    x = x_ref[...]
    gw = gw_ref[...]
    logits = lax.dot_general(x, gw, (((1,), (1,)), ((), ())),
                             preferred_element_type=jnp.float32)
    e_iota = lax.broadcasted_iota(jnp.int32, (T, E), 1)
    m1 = jnp.max(logits, axis=1, keepdims=True)
    idx1 = jnp.min(jnp.where(logits == m1, e_iota, E), axis=1, keepdims=True)
    masked = jnp.where(e_iota == idx1, -1e30, logits)
    m2 = jnp.max(masked, axis=1, keepdims=True)
    idx2 = jnp.min(jnp.where(masked == m2, e_iota, E), axis=1, keepdims=True)
    d = jnp.exp(m2 - m1)
    w0 = 1.0 / (1.0 + d)
    w1 = d / (1.0 + d)

    oh0 = (e_iota == idx1).astype(jnp.float32)
    oh1 = (e_iota == idx2).astype(jnp.float32)
    ohs = oh0 + oh1
    r_i = lax.broadcasted_iota(jnp.int32, (T, T), 0)
    c_i = lax.broadcasted_iota(jnp.int32, (T, T), 1)
    strict_lt = (c_i < r_i).astype(jnp.float32)
    rank = lax.dot_general(strict_lt, ohs, (((1,), (0,)), ((), ())),
                           preferred_element_type=jnp.float32)
    counts = rank[T - 1:T, :] + ohs[T - 1:T, :]
    padded = jnp.ceil(counts * (1.0 / BLK)) * BLK
    lt8_r = lax.broadcasted_iota(jnp.int32, (E, E), 0)
    lt8_c = lax.broadcasted_iota(jnp.int32, (E, E), 1)
    lt8 = (lt8_r < lt8_c).astype(jnp.float32)
    off = lax.dot_general(padded, lt8, (((1,), (0,)), ((), ())),
                          preferred_element_type=jnp.float32)
    off_end = off + padded

    slot0 = jnp.sum((off + rank) * oh0, axis=1)[None, :]
    slot1 = jnp.sum((off + rank) * oh1, axis=1)[None, :]
    slots_ref[...] = jnp.concatenate([slot0, slot1], axis=0).astype(jnp.int32)
    wts_ref[...] = jnp.concatenate([jnp.transpose(w0), jnp.transpose(w1)], axis=0)

    nb_i = (lax.broadcasted_iota(jnp.int32, (NB, E), 0) * BLK).astype(jnp.float32)
    be = jnp.sum((nb_i >= off_end).astype(jnp.int32), axis=1, keepdims=True)
    be = jnp.minimum(be, E - 1)

    r24 = lax.broadcasted_iota(jnp.int32, (NB, NB), 0)
    c24 = lax.broadcasted_iota(jnp.int32, (NB, NB), 1)
    be_r = jnp.broadcast_to(be, (NB, NB))
    be_c = jnp.broadcast_to(jnp.transpose(be), (NB, NB))
    tr_mat = jnp.where((c24 == r24 - 1) & (be_r != be_c), 1, 0)
    tr0 = jnp.sum(tr_mat, axis=1, keepdims=True)
    trc = jnp.broadcast_to(jnp.transpose(tr0), (NB, NB))
    tc = jnp.sum(jnp.where(c24 <= r24, trc, 0), axis=1, keepdims=True)
    parity = tc % 2
    ntm = jnp.where((c24 > r24) & (trc == 1), c24, NB)
    nt = jnp.min(ntm, axis=1, keepdims=True)
    nb_col = lax.broadcasted_iota(jnp.int32, (NB, 1), 0)
    isf = jnp.where((tr0 == 1) | (nb_col == 0), 1, 0)
    be_ref[...] = jnp.concatenate([be, parity, nt, isf], axis=1)


def _router(flat, gate_w):
    return pl.pallas_call(
        _router_body,
        out_shape=[
            jax.ShapeDtypeStruct((K, T), jnp.int32),
            jax.ShapeDtypeStruct((K, T), jnp.float32),
            jax.ShapeDtypeStruct((NB, 4), jnp.int32),
        ],
    )(flat, gate_w)


def _dispatch_body(flat_hbm, slots_hbm, wts_hbm,
                   xs_hbm, ws_hbm,
                   rows_v, idx0_v, idx1_v, slots_v, wts_v, wsort_v, sem):
    wid = lax.axis_index("s") * 2 + lax.axis_index("c")
    base = pl.multiple_of(wid * CT, CT)
    pltpu.sync_copy(flat_hbm.at[pl.ds(base, CT)], rows_v)
    pltpu.sync_copy(slots_hbm.at[0, pl.ds(base, CT)], idx0_v)
    pltpu.sync_copy(slots_hbm.at[1, pl.ds(base, CT)], idx1_v)
    cp0 = pltpu.async_copy(rows_v, xs_hbm.at[idx0_v], sem)
    cp1 = pltpu.async_copy(rows_v, xs_hbm.at[idx1_v], sem)

    @pl.when(wid == 0)
    def _():
        pltpu.sync_copy(slots_hbm, slots_v)
        pltpu.sync_copy(wts_hbm, wts_v)

        def zero(i, _):
            wsort_v[pl.ds(i * 16, 16)] = jnp.zeros((16,), jnp.float32)
            return 0
        lax.fori_loop(0, N_PAD // 16, zero, 0)

        def scat(i, _):
            k = i // (T // 16)
            j = i % (T // 16)
            sl = slots_v[k, pl.ds(j * 16, 16)]
            wv = wts_v[k, pl.ds(j * 16, 16)]
            plsc.store_scatter(wsort_v, [sl], wv)
            return 0
        lax.fori_loop(0, K * (T // 16), scat, 0)
        pltpu.sync_copy(wsort_v, ws_hbm)

    cp0.wait()
    cp1.wait()


def _dispatch(flat, slots, wts):
    mesh = plsc.VectorSubcoreMesh(core_axis_name="c", subcore_axis_name="s", num_cores=2, num_subcores=16)
    f = functools.partial(
        pl.kernel,
        out_type=[
            jax.ShapeDtypeStruct((N_PAD, H), jnp.float32),
            jax.ShapeDtypeStruct((N_PAD,), jnp.float32),
        ],
        mesh=mesh,
        scratch_types=[
            pltpu.VMEM((CT, H), jnp.float32),
            pltpu.VMEM((CT,), jnp.int32),
            pltpu.VMEM((CT,), jnp.int32),
            pltpu.VMEM((K, T), jnp.int32),
            pltpu.VMEM((K, T), jnp.float32),
            pltpu.VMEM((N_PAD,), jnp.float32),
            pltpu.SemaphoreType.DMA,
        ],
        compiler_params=pltpu.CompilerParams(needs_layout_passes=False),
    )(_dispatch_body)
    return f(flat, slots, wts)


def _ffn_body(be_ref, x_ref, demb_ref, w1_any, w3_any, w2_any, ws_ref, o_ref,
              w1b, w3b, w2b, sems):
    nb = pl.program_id(0)

    def copies(e, slot):
        return (
            pltpu.make_async_copy(w1_any.at[e], w1b.at[slot], sems.at[slot, 0]),
            pltpu.make_async_copy(w3_any.at[e], w3b.at[slot], sems.at[slot, 1]),
            pltpu.make_async_copy(w2_any.at[e], w2b.at[slot], sems.at[slot, 2]),
        )

    parity = be_ref[nb, 1]
    nt = be_ref[nb, 2]
    is_first = be_ref[nb, 3] == 1

    @pl.when(nb == 0)
    def _():
        for c in copies(be_ref[0, 0], 0):
            c.start()

    @pl.when(jnp.logical_and(is_first, nt < NB))
    def _():
        for c in copies(be_ref[jnp.minimum(nt, NB - 1), 0], (parity + 1) % 2):
            c.start()

    @pl.when(is_first)
    def _():
        for c in copies(be_ref[nb, 0], parity):
            c.wait()

    xb = x_ref[...] + demb_ref[0]
    h1 = lax.dot_general(xb, w1b[parity], (((1,), (1,)), ((), ())),
                         preferred_element_type=jnp.float32)
    h1 = h1 * jax.nn.sigmoid(h1)
    h2 = lax.dot_general(xb, w3b[parity], (((1,), (1,)), ((), ())),
                         preferred_element_type=jnp.float32)
    hh = h1 * h2
    part = lax.dot_general(hh, w2b[parity], (((1,), (1,)), ((), ())),
                           preferred_element_type=jnp.float32)
    w = ws_ref[0, 0, :][:, None]
    o_ref[...] = jnp.where(w == 0.0, 0.0, part * w)


def _ffn(block_expert, x_sorted, domain_emb, w1, w3, w2, w_sorted):
    ws3 = w_sorted.reshape(NB, 1, BLK)
    grid_spec = pltpu.PrefetchScalarGridSpec(
        num_scalar_prefetch=1,
        grid=(NB,),
        in_specs=[
            pl.BlockSpec((BLK, H), lambda nb, be: (nb, 0)),
            pl.BlockSpec((1, 1, H), lambda nb, be: (be[nb, 0], 0, 0)),
            pl.BlockSpec(memory_space=pl.ANY),
            pl.BlockSpec(memory_space=pl.ANY),
            pl.BlockSpec(memory_space=pl.ANY),
            pl.BlockSpec((1, 1, BLK), lambda nb, be: (nb, 0, 0)),
        ],
        out_specs=pl.BlockSpec((BLK, H), lambda nb, be: (nb, 0)),
        scratch_shapes=[
            pltpu.VMEM((2, FF, H), jnp.float32),
            pltpu.VMEM((2, FF, H), jnp.float32),
            pltpu.VMEM((2, H, FF), jnp.float32),
            pltpu.SemaphoreType.DMA((2, 3)),
        ],
    )
    return pl.pallas_call(
        _ffn_body,
        grid_spec=grid_spec,
        out_shape=jax.ShapeDtypeStruct((N_PAD, H), jnp.float32),
        compiler_params=pltpu.CompilerParams(vmem_limit_bytes=120 * 1024 * 1024),
    )(block_expert, x_sorted, domain_emb.reshape(E, 1, H), w1, w3, w2, ws3)


NCH = 4
CR = CT // NCH


def _combine_body(y_hbm, slots_hbm, out_hbm, y0_v, y1_v, i0_v, i1_v, sems):
    wid = lax.axis_index("s") * 2 + lax.axis_index("c")
    base = pl.multiple_of(wid * CT, CT)
    pltpu.sync_copy(slots_hbm.at[0, pl.ds(base, CT)], i0_v)
    pltpu.sync_copy(slots_hbm.at[1, pl.ds(base, CT)], i1_v)
    cps = []
    for c in range(NCH):
        sl = pl.ds(c * CR, CR)
        cps.append(pltpu.async_copy(
            y_hbm.at[i0_v.at[sl]], y0_v.at[sl], sems.at[c, 0]))
        cps.append(pltpu.async_copy(
            y_hbm.at[i1_v.at[sl]], y1_v.at[sl], sems.at[c, 1]))
    stores = []
    for c in range(NCH):
        cps[2 * c].wait()
        cps[2 * c + 1].wait()

        def add(i, _):
            r = c * CR + i // (H // 64)
            col = (i % (H // 64)) * 64
            for u in range(4):
                s = pl.ds(col + u * 16, 16)
                y0_v[r, s] = y0_v[r, s] + y1_v[r, s]
            return 0
        lax.fori_loop(0, CR * (H // 64), add, 0)
        sl = pl.ds(c * CR, CR)
        stores.append(pltpu.async_copy(
            y0_v.at[sl], out_hbm.at[pl.ds(base + c * CR, CR)], sems.at[c, 2]))
    for st in stores:
        st.wait()


def _combine(y, slots):
    mesh = plsc.VectorSubcoreMesh(core_axis_name="c", subcore_axis_name="s", num_cores=2, num_subcores=16)
    f = functools.partial(
        pl.kernel,
        out_type=jax.ShapeDtypeStruct((T, H), jnp.float32),
        mesh=mesh,
        scratch_types=[
            pltpu.VMEM((CT, H), jnp.float32),
            pltpu.VMEM((CT, H), jnp.float32),
            pltpu.VMEM((CT,), jnp.int32),
            pltpu.VMEM((CT,), jnp.int32),
            pltpu.SemaphoreType.DMA((NCH, 3)),
        ],
        compiler_params=pltpu.CompilerParams(needs_layout_passes=False),
    )(_combine_body)
    return f(y, slots)


def kernel(hidden_states, gate_w, w1, w2, w3, domain_emb):
    b, s, h = hidden_states.shape
    flat = hidden_states.reshape(T, H)
    slots, wts, be = _router(flat, gate_w)
    x_sorted, w_sorted = _dispatch(flat, slots, wts)
    y = _ffn(be, x_sorted, domain_emb, w1, w3, w2, w_sorted)
    out = _combine(y, slots)
    return out.reshape(b, s, h)


# speedup vs baseline: 1.0408x; 1.0190x over previous
import functools

import jax
import jax.numpy as jnp
from jax import lax
from jax.experimental import pallas as pl
from jax.experimental.pallas import tpu as pltpu
from jax.experimental.pallas import tpu_sc as plsc

T = 2048
H = 768
FF = 3072
E = 8
K = 2

BLK = 256
NB = (T * K + E * BLK) // BLK
N_PAD = NB * BLK
FF_BLK = 768
NF = FF // FF_BLK

NW = 32
CT = T // NW


def _router_body(x_ref, gw_ref, slots_ref, wts_ref, be_ref):
    x = x_ref[...]
    gw = gw_ref[...]
    logits = lax.dot_general(x, gw, (((1,), (1,)), ((), ())),
                             preferred_element_type=jnp.float32)
    e_iota = lax.broadcasted_iota(jnp.int32, (T, E), 1)
    m1 = jnp.max(logits, axis=1, keepdims=True)
    idx1 = jnp.min(jnp.where(logits == m1, e_iota, E), axis=1, keepdims=True)
    masked = jnp.where(e_iota == idx1, -1e30, logits)
    m2 = jnp.max(masked, axis=1, keepdims=True)
    idx2 = jnp.min(jnp.where(masked == m2, e_iota, E), axis=1, keepdims=True)
    d = jnp.exp(m2 - m1)
    w0 = 1.0 / (1.0 + d)
    w1 = d / (1.0 + d)

    oh0 = (e_iota == idx1).astype(jnp.float32)
    oh1 = (e_iota == idx2).astype(jnp.float32)
    ohs = oh0 + oh1
    r_i = lax.broadcasted_iota(jnp.int32, (T, T), 0)
    c_i = lax.broadcasted_iota(jnp.int32, (T, T), 1)
    strict_lt = (c_i < r_i).astype(jnp.float32)
    rank = lax.dot_general(strict_lt, ohs, (((1,), (0,)), ((), ())),
                           preferred_element_type=jnp.float32)
    counts = rank[T - 1:T, :] + ohs[T - 1:T, :]
    padded = jnp.ceil(counts * (1.0 / BLK)) * BLK
    lt8_r = lax.broadcasted_iota(jnp.int32, (E, E), 0)
    lt8_c = lax.broadcasted_iota(jnp.int32, (E, E), 1)
    lt8 = (lt8_r < lt8_c).astype(jnp.float32)
    off = lax.dot_general(padded, lt8, (((1,), (0,)), ((), ())),
                          preferred_element_type=jnp.float32)
    off_end = off + padded

    slot0 = jnp.sum((off + rank) * oh0, axis=1)[None, :]
    slot1 = jnp.sum((off + rank) * oh1, axis=1)[None, :]
    slots_ref[...] = jnp.concatenate([slot0, slot1], axis=0).astype(jnp.int32)
    wts_ref[...] = jnp.concatenate([jnp.transpose(w0), jnp.transpose(w1)], axis=0)

    nb_i = (lax.broadcasted_iota(jnp.int32, (NB, E), 0) * BLK).astype(jnp.float32)
    be = jnp.sum((nb_i >= off_end).astype(jnp.int32), axis=1, keepdims=True)
    be = jnp.minimum(be, E - 1)

    r24 = lax.broadcasted_iota(jnp.int32, (NB, NB), 0)
    c24 = lax.broadcasted_iota(jnp.int32, (NB, NB), 1)
    be_r = jnp.broadcast_to(be, (NB, NB))
    be_c = jnp.broadcast_to(jnp.transpose(be), (NB, NB))
    tr_mat = jnp.where((c24 == r24 - 1) & (be_r != be_c), 1, 0)
    tr0 = jnp.sum(tr_mat, axis=1, keepdims=True)
    trc = jnp.broadcast_to(jnp.transpose(tr0), (NB, NB))
    tc = jnp.sum(jnp.where(c24 <= r24, trc, 0), axis=1, keepdims=True)
    parity = tc % 2
    ntm = jnp.where((c24 > r24) & (trc == 1), c24, NB)
    nt = jnp.min(ntm, axis=1, keepdims=True)
    nb_col = lax.broadcasted_iota(jnp.int32, (NB, 1), 0)
    isf = jnp.where((tr0 == 1) | (nb_col == 0), 1, 0)
    be_ref[...] = jnp.concatenate([be, parity, nt, isf], axis=1)


def _router(flat, gate_w):
    return pl.pallas_call(
        _router_body,
        out_shape=[
            jax.ShapeDtypeStruct((K, T), jnp.int32),
            jax.ShapeDtypeStruct((K, T), jnp.float32),
            jax.ShapeDtypeStruct((NB, 4), jnp.int32),
        ],
    )(flat, gate_w)


def _dispatch_body(flat_hbm, slots_hbm, wts_hbm,
                   xs_hbm, ws_hbm,
                   rows_v, idx0_v, idx1_v, slots_v, wts_v, wsort_v, sem):
    wid = lax.axis_index("s") * 2 + lax.axis_index("c")
    base = pl.multiple_of(wid * CT, CT)
    pltpu.sync_copy(flat_hbm.at[pl.ds(base, CT)], rows_v)
    pltpu.sync_copy(slots_hbm.at[0, pl.ds(base, CT)], idx0_v)
    pltpu.sync_copy(slots_hbm.at[1, pl.ds(base, CT)], idx1_v)
    cp0 = pltpu.async_copy(rows_v, xs_hbm.at[idx0_v], sem)
    cp1 = pltpu.async_copy(rows_v, xs_hbm.at[idx1_v], sem)

    @pl.when(wid == 0)
    def _():
        pltpu.sync_copy(slots_hbm, slots_v)
        pltpu.sync_copy(wts_hbm, wts_v)

        def zero(i, _):
            wsort_v[pl.ds(i * 16, 16)] = jnp.zeros((16,), jnp.float32)
            return 0
        lax.fori_loop(0, N_PAD // 16, zero, 0)

        def scat(i, _):
            k = i // (T // 16)
            j = i % (T // 16)
            sl = slots_v[k, pl.ds(j * 16, 16)]
            wv = wts_v[k, pl.ds(j * 16, 16)]
            plsc.store_scatter(wsort_v, [sl], wv)
            return 0
        lax.fori_loop(0, K * (T // 16), scat, 0)
        pltpu.sync_copy(wsort_v, ws_hbm)

    cp0.wait()
    cp1.wait()


def _dispatch(flat, slots, wts):
    mesh = plsc.VectorSubcoreMesh(core_axis_name="c", subcore_axis_name="s", num_cores=2, num_subcores=16)
    f = functools.partial(
        pl.kernel,
        out_type=[
            jax.ShapeDtypeStruct((N_PAD, H), jnp.float32),
            jax.ShapeDtypeStruct((N_PAD,), jnp.float32),
        ],
        mesh=mesh,
        scratch_types=[
            pltpu.VMEM((CT, H), jnp.float32),
            pltpu.VMEM((CT,), jnp.int32),
            pltpu.VMEM((CT,), jnp.int32),
            pltpu.VMEM((K, T), jnp.int32),
            pltpu.VMEM((K, T), jnp.float32),
            pltpu.VMEM((N_PAD,), jnp.float32),
            pltpu.SemaphoreType.DMA,
        ],
        compiler_params=pltpu.CompilerParams(needs_layout_passes=False),
    )(_dispatch_body)
    return f(flat, slots, wts)


def _ffn_body(be_ref, x_ref, demb_ref, w1_any, w3_any, w2_any, ws_ref, o_ref,
              w1b, w3b, w2b, sems):
    nb = pl.program_id(0)

    def copies(e, slot):
        return (
            pltpu.make_async_copy(w1_any.at[e], w1b.at[slot], sems.at[slot, 0]),
            pltpu.make_async_copy(w3_any.at[e], w3b.at[slot], sems.at[slot, 1]),
            pltpu.make_async_copy(w2_any.at[e], w2b.at[slot], sems.at[slot, 2]),
        )

    parity = be_ref[nb, 1]
    nt = be_ref[nb, 2]
    is_first = be_ref[nb, 3] == 1

    @pl.when(nb == 0)
    def _():
        for c in copies(be_ref[0, 0], 0):
            c.start()

    @pl.when(jnp.logical_and(is_first, nt < NB))
    def _():
        for c in copies(be_ref[jnp.minimum(nt, NB - 1), 0], (parity + 1) % 2):
            c.start()

    @pl.when(is_first)
    def _():
        for c in copies(be_ref[nb, 0], parity):
            c.wait()

    xb = x_ref[...] + demb_ref[0]
    h1 = lax.dot_general(xb, w1b[parity], (((1,), (1,)), ((), ())),
                         preferred_element_type=jnp.float32)
    h1 = h1 * jax.nn.sigmoid(h1)
    h2 = lax.dot_general(xb, w3b[parity], (((1,), (1,)), ((), ())),
                         preferred_element_type=jnp.float32)
    hh = h1 * h2
    part = lax.dot_general(hh, w2b[parity], (((1,), (1,)), ((), ())),
                           preferred_element_type=jnp.float32)
    w = ws_ref[0, 0, :][:, None]
    o_ref[...] = jnp.where(w == 0.0, 0.0, part * w)


def _ffn(block_expert, x_sorted, domain_emb, w1, w3, w2, w_sorted):
    ws3 = w_sorted.reshape(NB, 1, BLK)
    grid_spec = pltpu.PrefetchScalarGridSpec(
        num_scalar_prefetch=1,
        grid=(NB,),
        in_specs=[
            pl.BlockSpec((BLK, H), lambda nb, be: (nb, 0)),
            pl.BlockSpec((1, 1, H), lambda nb, be: (be[nb, 0], 0, 0)),
            pl.BlockSpec(memory_space=pl.ANY),
            pl.BlockSpec(memory_space=pl.ANY),
            pl.BlockSpec(memory_space=pl.ANY),
            pl.BlockSpec((1, 1, BLK), lambda nb, be: (nb, 0, 0)),
        ],
        out_specs=pl.BlockSpec((BLK, H), lambda nb, be: (nb, 0)),
        scratch_shapes=[
            pltpu.VMEM((2, FF, H), jnp.float32),
            pltpu.VMEM((2, FF, H), jnp.float32),
            pltpu.VMEM((2, H, FF), jnp.float32),
            pltpu.SemaphoreType.DMA((2, 3)),
        ],
    )
    return pl.pallas_call(
        _ffn_body,
        grid_spec=grid_spec,
        out_shape=jax.ShapeDtypeStruct((N_PAD, H), jnp.float32),
        compiler_params=pltpu.CompilerParams(vmem_limit_bytes=120 * 1024 * 1024),
    )(block_expert, x_sorted, domain_emb.reshape(E, 1, H), w1, w3, w2, ws3)


NCH = 8
CR = CT // NCH


def _combine_body(y_hbm, slots_hbm, out_hbm, y0_v, y1_v, i0_v, i1_v, sems):
    wid = lax.axis_index("s") * 2 + lax.axis_index("c")
    base = pl.multiple_of(wid * CT, CT)
    pltpu.sync_copy(slots_hbm.at[0, pl.ds(base, CT)], i0_v)
    pltpu.sync_copy(slots_hbm.at[1, pl.ds(base, CT)], i1_v)
    cps = []
    for c in range(NCH):
        sl = pl.ds(c * CR, CR)
        cps.append(pltpu.async_copy(
            y_hbm.at[i0_v.at[sl]], y0_v.at[sl], sems.at[c, 0]))
        cps.append(pltpu.async_copy(
            y_hbm.at[i1_v.at[sl]], y1_v.at[sl], sems.at[c, 1]))
    stores = []
    for c in range(NCH):
        cps[2 * c].wait()
        cps[2 * c + 1].wait()

        def add(i, _):
            r = c * CR + i // (H // 64)
            col = (i % (H // 64)) * 64
            for u in range(4):
                s = pl.ds(col + u * 16, 16)
                y0_v[r, s] = y0_v[r, s] + y1_v[r, s]
            return 0
        lax.fori_loop(0, CR * (H // 64), add, 0)
        sl = pl.ds(c * CR, CR)
        stores.append(pltpu.async_copy(
            y0_v.at[sl], out_hbm.at[pl.ds(base + c * CR, CR)], sems.at[c, 2]))
    for st in stores:
        st.wait()


def _combine(y, slots):
    mesh = plsc.VectorSubcoreMesh(core_axis_name="c", subcore_axis_name="s", num_cores=2, num_subcores=16)
    f = functools.partial(
        pl.kernel,
        out_type=jax.ShapeDtypeStruct((T, H), jnp.float32),
        mesh=mesh,
        scratch_types=[
            pltpu.VMEM((CT, H), jnp.float32),
            pltpu.VMEM((CT, H), jnp.float32),
            pltpu.VMEM((CT,), jnp.int32),
            pltpu.VMEM((CT,), jnp.int32),
            pltpu.SemaphoreType.DMA((NCH, 3)),
        ],
        compiler_params=pltpu.CompilerParams(needs_layout_passes=False),
    )(_combine_body)
    return f(y, slots)


def kernel(hidden_states, gate_w, w1, w2, w3, domain_emb):
    b, s, h = hidden_states.shape
    flat = hidden_states.reshape(T, H)
    slots, wts, be = _router(flat, gate_w)
    x_sorted, w_sorted = _dispatch(flat, slots, wts)
    y = _ffn(be, x_sorted, domain_emb, w1, w3, w2, w_sorted)
    out = _combine(y, slots)
    return out.reshape(b, s, h)
